# Initial kernel scaffold; baseline (speedup 1.0000x reference)
import jax
import jax.numpy as jnp
from jax.experimental import pallas as pl


def kernel(fnode, fmess, agraph, bgraph, unused, atom_scope, bond_scope, W_i, W_z, U_z, W_r, U_r, W_h, U_h, W_o, W_ff1, W_ff2, ln_g, ln_b):
    raise NotImplementedError("write your pallas kernel here")


# speedup vs baseline: 33.5832x; 33.5832x over previous
import functools

import jax
import jax.numpy as jnp
from jax import lax
from jax.experimental import pallas as pl
from jax.experimental.pallas import tpu as pltpu
from jax.experimental.pallas import tpu_sc as plsc

N_NODES = 10000
N_EDGES = 160000
MAX_NB = 16
NODE_FDIM = 128
EDGE_FDIM = 16
FMESS_DIM = NODE_FDIM + EDGE_FDIM
HSIZE = 128
N_HEADS = 4
HEAD = HSIZE // N_HEADS
DEPTH = 3
N_BLOCKS = 2
N_MOL = 100
A_PER = N_NODES // N_MOL

_F32 = jnp.float32
_NW = 32


@functools.lru_cache(maxsize=None)
def _make_gather_sum(n_out: int, chunk: int):
    epw = n_out // _NW
    assert epw * _NW == n_out and epw % chunk == 0
    n_chunks = epw // chunk
    mesh = plsc.VectorSubcoreMesh(core_axis_name="c", subcore_axis_name="s")

    @functools.partial(
        pl.kernel,
        mesh=mesh,
        out_type=jax.ShapeDtypeStruct((n_out, HSIZE), _F32),
        scratch_types=[
            pltpu.VMEM((chunk * MAX_NB,), jnp.int32),
            pltpu.VMEM((chunk * MAX_NB, HSIZE), _F32),
            pltpu.VMEM((chunk, HSIZE), _F32),
            pltpu.SemaphoreType.DMA,
        ],
    )
    def gsum(table_hbm, idx_hbm, out_hbm, idx_v, rows_v, acc_v, sem):
        wid = lax.axis_index("s") * 2 + lax.axis_index("c")
        base = wid * epw

        def chunk_body(c, carry):
            e0 = base + c * chunk
            pltpu.sync_copy(idx_hbm.at[pl.ds(e0 * MAX_NB, chunk * MAX_NB)], idx_v)
            pltpu.async_copy(table_hbm.at[idx_v], rows_v, sem).wait()

            def edge_body(e, carry2):
                r0 = e * MAX_NB
                for s in range(HSIZE // 16):
                    col = pl.ds(s * 16, 16)
                    acc = rows_v[r0, col]
                    for j in range(1, MAX_NB):
                        acc = acc + rows_v[r0 + j, col]
                    acc_v[e, col] = acc
                return carry2

            lax.fori_loop(0, chunk, edge_body, 0, unroll=False)
            pltpu.sync_copy(acc_v, out_hbm.at[pl.ds(e0, chunk)])
            return carry

        lax.fori_loop(0, n_chunks, chunk_body, 0, unroll=False)

    return gsum


E_PAD = 163840
A_PAD = 10240


def _gsum_edges(table, idx):
    return _make_gather_sum(E_PAD, 16)(table, idx)


def _gsum_atoms(table, idx):
    return _make_gather_sum(A_PAD, 16)(table, idx)


def _dot(a, b):
    return jnp.dot(a, b, preferred_element_type=_F32)


def _init_body(fmess_ref, wi_ref, wzrh_ref, x_ref, p_ref):
    x = jnp.maximum(_dot(fmess_ref[...], wi_ref[...]), 0.0)
    x_ref[...] = x
    p_ref[...] = _dot(x, wzrh_ref[...])


def _mpn_init(fmess, wi_cat, wzrh, rowb=2048):
    n = fmess.shape[0]
    grid = n // rowb
    return pl.pallas_call(
        _init_body,
        grid=(grid,),
        in_specs=[
            pl.BlockSpec((rowb, FMESS_DIM), lambda i: (i, 0)),
            pl.BlockSpec((FMESS_DIM, HSIZE), lambda i: (0, 0)),
            pl.BlockSpec((HSIZE, 3 * HSIZE), lambda i: (0, 0)),
        ],
        out_specs=[
            pl.BlockSpec((rowb, HSIZE), lambda i: (i, 0)),
            pl.BlockSpec((rowb, 3 * HSIZE), lambda i: (i, 0)),
        ],
        out_shape=[
            jax.ShapeDtypeStruct((n, HSIZE), _F32),
            jax.ShapeDtypeStruct((n, 3 * HSIZE), _F32),
        ],
    )(fmess, wi_cat, wzrh)


def _gru_body(s_ref, p_ref, uzr_ref, uh_ref, m_ref):
    s = s_ref[...]
    p = p_ref[...]
    q = _dot(s, uzr_ref[...])
    z = jax.nn.sigmoid(p[:, :HSIZE] + q[:, :HSIZE])
    r = jax.nn.sigmoid(p[:, HSIZE:2 * HSIZE] + q[:, HSIZE:])
    mt = jnp.tanh(p[:, 2 * HSIZE:] + _dot(r * s, uh_ref[...]))
    m_ref[...] = (1.0 - z) * s + z * mt


def _gru_update(s, p, uzr, uh, rowb=2048):
    n = s.shape[0]
    grid = n // rowb
    return pl.pallas_call(
        _gru_body,
        grid=(grid,),
        in_specs=[
            pl.BlockSpec((rowb, HSIZE), lambda i: (i, 0)),
            pl.BlockSpec((rowb, 3 * HSIZE), lambda i: (i, 0)),
            pl.BlockSpec((HSIZE, 2 * HSIZE), lambda i: (0, 0)),
            pl.BlockSpec((HSIZE, HSIZE), lambda i: (0, 0)),
        ],
        out_specs=pl.BlockSpec((rowb, HSIZE), lambda i: (i, 0)),
        out_shape=jax.ShapeDtypeStruct((n, HSIZE), _F32),
    )(s, p, uzr, uh)


def _outproj_body(hnode_ref, nei_ref, wot_ref, wob_ref, out_ref):
    out_ref[...] = jnp.maximum(
        _dot(hnode_ref[...], wot_ref[...]) + _dot(nei_ref[...], wob_ref[...]), 0.0)


def _out_proj(hnode, nei, wot, wob, rowb=2000):
    n = hnode.shape[0]
    grid = n // rowb
    return pl.pallas_call(
        _outproj_body,
        grid=(grid,),
        in_specs=[
            pl.BlockSpec((rowb, HSIZE), lambda i: (i, 0)),
            pl.BlockSpec((rowb, HSIZE), lambda i: (i, 0)),
            pl.BlockSpec((HSIZE, HSIZE), lambda i: (0, 0)),
            pl.BlockSpec((HSIZE, HSIZE), lambda i: (0, 0)),
        ],
        out_specs=pl.BlockSpec((rowb, HSIZE), lambda i: (i, 0)),
        out_shape=jax.ShapeDtypeStruct((n, HSIZE), _F32),
    )(hnode, nei, wot, wob)


_FF_ROWB = 2000
_FF_MOLB = _FF_ROWB // A_PER


def _ffn_body(hnode_ref, fnode_ref, w1_ref, w2_ref, g_ref, b_ref,
              hatom_ref, hmol_ref):
    hcat = jnp.concatenate([hnode_ref[...], fnode_ref[...]], axis=1)
    h1 = jnp.maximum(_dot(hcat, w1_ref[...]), 0.0)
    hmid = _dot(h1, w2_ref[...])
    mu = jnp.mean(hmid, axis=1, keepdims=True)
    var = jnp.mean((hmid - mu) ** 2, axis=1, keepdims=True)
    hatom = (hmid - mu) * lax.rsqrt(var + 1e-5) * g_ref[...] + b_ref[...]
    hatom_ref[...] = hatom
    rows = lax.broadcasted_iota(jnp.int32, (_FF_MOLB, _FF_ROWB), 1) // A_PER
    mols = lax.broadcasted_iota(jnp.int32, (_FF_MOLB, _FF_ROWB), 0)
    ind = jnp.where(rows == mols, 1.0, 0.0).astype(_F32)
    hmol_ref[...] = _dot(ind, hatom)[None]


def _ffn(hnode, fnode, w1, w2, g, b):
    n = hnode.shape[0]
    grid = n // _FF_ROWB
    return pl.pallas_call(
        _ffn_body,
        grid=(grid,),
        in_specs=[
            pl.BlockSpec((_FF_ROWB, HSIZE), lambda i: (i, 0)),
            pl.BlockSpec((_FF_ROWB, NODE_FDIM), lambda i: (i, 0)),
            pl.BlockSpec((HSIZE + NODE_FDIM, 2 * HSIZE), lambda i: (0, 0)),
            pl.BlockSpec((2 * HSIZE, HSIZE), lambda i: (0, 0)),
            pl.BlockSpec((1, HSIZE), lambda i: (0, 0)),
            pl.BlockSpec((1, HSIZE), lambda i: (0, 0)),
        ],
        out_specs=[
            pl.BlockSpec((_FF_ROWB, HSIZE), lambda i: (i, 0)),
            pl.BlockSpec((1, _FF_MOLB, HSIZE), lambda i: (i, 0, 0)),
        ],
        out_shape=[
            jax.ShapeDtypeStruct((n, HSIZE), _F32),
            jax.ShapeDtypeStruct((grid, _FF_MOLB, HSIZE), _F32),
        ],
    )(hnode, fnode, w1, w2, g, b)


def _blockdiag(w):
    h, a, b = w.shape
    out = jnp.zeros((h * a, h * b), w.dtype)
    for i in range(h):
        out = out.at[i * a:(i + 1) * a, i * b:(i + 1) * b].set(w[i])
    return out


def kernel(fnode, fmess, agraph, bgraph, unused, atom_scope, bond_scope,
           W_i, W_z, U_z, W_r, U_r, W_h, U_h, W_o, W_ff1, W_ff2, ln_g, ln_b):
    bidx = jnp.pad(bgraph.reshape(-1).astype(jnp.int32),
                   (0, (E_PAD - N_EDGES) * MAX_NB))
    aidx = jnp.pad(agraph.reshape(-1).astype(jnp.int32),
                   (0, (A_PAD - N_NODES) * MAX_NB))
    fmess = jnp.pad(fmess, ((0, E_PAD - N_EDGES), (0, 0)))

    hnode = fnode
    for b in range(N_BLOCKS):
        wi_cat = jnp.concatenate([W_i[b, h] for h in range(N_HEADS)], axis=1)
        wzrh = jnp.concatenate(
            [_blockdiag(W_z[b]), _blockdiag(W_r[b]), _blockdiag(W_h[b])], axis=1)
        uzr = jnp.concatenate([_blockdiag(U_z[b]), _blockdiag(U_r[b])], axis=1)
        uh = _blockdiag(U_h[b])

        x, p = _mpn_init(fmess, wi_cat, wzrh)
        m = x
        for _ in range(DEPTH - 1):
            s_nei = _gsum_edges(m, bidx)
            m = _gru_update(s_nei, p, uzr, uh)

        nei = _gsum_atoms(m, aidx)[:N_NODES]
        wot = jnp.concatenate([W_o[b, h][:HSIZE] for h in range(N_HEADS)], axis=1)
        wob = _blockdiag(W_o[b, :, HSIZE:, :])
        hnode = _out_proj(hnode, nei, wot, wob)

    hatom, hmol = _ffn(hnode, fnode, W_ff1, W_ff2,
                       ln_g.reshape(1, -1), ln_b.reshape(1, -1))
    return (hmol.reshape(N_MOL, HSIZE), hatom)


# speedup vs baseline: 38.3584x; 1.1422x over previous
import functools

import jax
import jax.numpy as jnp
from jax import lax
from jax.experimental import pallas as pl
from jax.experimental.pallas import tpu as pltpu
from jax.experimental.pallas import tpu_sc as plsc

N_NODES = 10000
N_EDGES = 160000
MAX_NB = 16
NODE_FDIM = 128
EDGE_FDIM = 16
FMESS_DIM = NODE_FDIM + EDGE_FDIM
HSIZE = 128
N_HEADS = 4
HEAD = HSIZE // N_HEADS
DEPTH = 3
N_BLOCKS = 2
N_MOL = 100
A_PER = N_NODES // N_MOL

_F32 = jnp.float32
_NW = 32


_CH = 16
_NCH = 4
_SUP = _CH * _NCH


@functools.lru_cache(maxsize=None)
def _make_gather_sum(n_out: int):
    epw = n_out // _NW
    assert epw * _NW == n_out and epw % _SUP == 0
    n_sup = epw // _SUP
    mesh = plsc.VectorSubcoreMesh(core_axis_name="c", subcore_axis_name="s")

    @functools.partial(
        pl.kernel,
        mesh=mesh,
        out_type=jax.ShapeDtypeStruct((n_out, HSIZE), _F32),
        scratch_types=[
            pltpu.VMEM((_SUP * MAX_NB,), jnp.int32),
            pltpu.VMEM((_CH * MAX_NB, HSIZE), _F32),
            pltpu.VMEM((_CH * MAX_NB, HSIZE), _F32),
            pltpu.VMEM((_SUP, HSIZE), _F32),
            pltpu.SemaphoreType.DMA,
            pltpu.SemaphoreType.DMA,
        ],
    )
    def gsum(table_hbm, idx_hbm, out_hbm, idx_v, rows0, rows1, acc_v, sem0, sem1):
        wid = lax.axis_index("s") * 2 + lax.axis_index("c")
        base = wid * epw
        rows = [rows0, rows1]
        sems = [sem0, sem1]

        def sup_body(sp, carry):
            e0 = base + sp * _SUP
            pltpu.sync_copy(idx_hbm.at[pl.ds(e0 * MAX_NB, _SUP * MAX_NB)], idx_v)
            cps = [None, None]
            cps[0] = pltpu.async_copy(
                table_hbm.at[idx_v.at[pl.ds(0, _CH * MAX_NB)]], rows[0], sems[0])
            for c in range(_NCH):
                pb = c % 2
                if c + 1 < _NCH:
                    cps[1 - pb] = pltpu.async_copy(
                        table_hbm.at[idx_v.at[pl.ds((c + 1) * _CH * MAX_NB,
                                                    _CH * MAX_NB)]],
                        rows[1 - pb], sems[1 - pb])
                cps[pb].wait()
                rv = rows[pb]

                def edge_body(e, carry2, _c=c, _rv=rv):
                    r0 = e * MAX_NB
                    for s in range(HSIZE // 16):
                        col = pl.ds(s * 16, 16)
                        t = [_rv[r0 + 2 * j, col] + _rv[r0 + 2 * j + 1, col]
                             for j in range(8)]
                        u = [t[2 * j] + t[2 * j + 1] for j in range(4)]
                        acc_v[_c * _CH + e, col] = (u[0] + u[1]) + (u[2] + u[3])
                    return carry2

                lax.fori_loop(0, _CH, edge_body, 0, unroll=False)
            pltpu.sync_copy(acc_v, out_hbm.at[pl.ds(e0, _SUP)])
            return carry

        lax.fori_loop(0, n_sup, sup_body, 0, unroll=False)

    return gsum


E_PAD = 163840
A_PAD = 10240


def _gsum_edges(table, idx):
    return _make_gather_sum(E_PAD)(table, idx)


def _gsum_atoms(table, idx):
    return _make_gather_sum(A_PAD)(table, idx)


def _dot(a, b):
    return jnp.dot(a, b, preferred_element_type=_F32)


def _init_body(fmess_ref, wi_ref, wzrh_ref, x_ref, p_ref):
    x = jnp.maximum(_dot(fmess_ref[...], wi_ref[...]), 0.0)
    x_ref[...] = x
    p_ref[...] = _dot(x, wzrh_ref[...])


def _mpn_init(fmess, wi_cat, wzrh, rowb=2048):
    n = fmess.shape[0]
    grid = n // rowb
    return pl.pallas_call(
        _init_body,
        grid=(grid,),
        in_specs=[
            pl.BlockSpec((rowb, FMESS_DIM), lambda i: (i, 0)),
            pl.BlockSpec((FMESS_DIM, HSIZE), lambda i: (0, 0)),
            pl.BlockSpec((HSIZE, 3 * HSIZE), lambda i: (0, 0)),
        ],
        out_specs=[
            pl.BlockSpec((rowb, HSIZE), lambda i: (i, 0)),
            pl.BlockSpec((rowb, 3 * HSIZE), lambda i: (i, 0)),
        ],
        out_shape=[
            jax.ShapeDtypeStruct((n, HSIZE), _F32),
            jax.ShapeDtypeStruct((n, 3 * HSIZE), _F32),
        ],
    )(fmess, wi_cat, wzrh)


def _gru_body(s_ref, p_ref, uzr_ref, uh_ref, m_ref):
    s = s_ref[...]
    p = p_ref[...]
    q = _dot(s, uzr_ref[...])
    z = jax.nn.sigmoid(p[:, :HSIZE] + q[:, :HSIZE])
    r = jax.nn.sigmoid(p[:, HSIZE:2 * HSIZE] + q[:, HSIZE:])
    mt = jnp.tanh(p[:, 2 * HSIZE:] + _dot(r * s, uh_ref[...]))
    m_ref[...] = (1.0 - z) * s + z * mt


def _gru_update(s, p, uzr, uh, rowb=2048):
    n = s.shape[0]
    grid = n // rowb
    return pl.pallas_call(
        _gru_body,
        grid=(grid,),
        in_specs=[
            pl.BlockSpec((rowb, HSIZE), lambda i: (i, 0)),
            pl.BlockSpec((rowb, 3 * HSIZE), lambda i: (i, 0)),
            pl.BlockSpec((HSIZE, 2 * HSIZE), lambda i: (0, 0)),
            pl.BlockSpec((HSIZE, HSIZE), lambda i: (0, 0)),
        ],
        out_specs=pl.BlockSpec((rowb, HSIZE), lambda i: (i, 0)),
        out_shape=jax.ShapeDtypeStruct((n, HSIZE), _F32),
    )(s, p, uzr, uh)


def _outproj_body(hnode_ref, nei_ref, wot_ref, wob_ref, out_ref):
    out_ref[...] = jnp.maximum(
        _dot(hnode_ref[...], wot_ref[...]) + _dot(nei_ref[...], wob_ref[...]), 0.0)


def _out_proj(hnode, nei, wot, wob, rowb=2000):
    n = hnode.shape[0]
    grid = n // rowb
    return pl.pallas_call(
        _outproj_body,
        grid=(grid,),
        in_specs=[
            pl.BlockSpec((rowb, HSIZE), lambda i: (i, 0)),
            pl.BlockSpec((rowb, HSIZE), lambda i: (i, 0)),
            pl.BlockSpec((HSIZE, HSIZE), lambda i: (0, 0)),
            pl.BlockSpec((HSIZE, HSIZE), lambda i: (0, 0)),
        ],
        out_specs=pl.BlockSpec((rowb, HSIZE), lambda i: (i, 0)),
        out_shape=jax.ShapeDtypeStruct((n, HSIZE), _F32),
    )(hnode, nei, wot, wob)


_FF_ROWB = 2000
_FF_MOLB = _FF_ROWB // A_PER


def _ffn_body(hnode_ref, fnode_ref, w1_ref, w2_ref, g_ref, b_ref,
              hatom_ref, hmol_ref):
    hcat = jnp.concatenate([hnode_ref[...], fnode_ref[...]], axis=1)
    h1 = jnp.maximum(_dot(hcat, w1_ref[...]), 0.0)
    hmid = _dot(h1, w2_ref[...])
    mu = jnp.mean(hmid, axis=1, keepdims=True)
    var = jnp.mean((hmid - mu) ** 2, axis=1, keepdims=True)
    hatom = (hmid - mu) * lax.rsqrt(var + 1e-5) * g_ref[...] + b_ref[...]
    hatom_ref[...] = hatom
    rows = lax.broadcasted_iota(jnp.int32, (_FF_MOLB, _FF_ROWB), 1) // A_PER
    mols = lax.broadcasted_iota(jnp.int32, (_FF_MOLB, _FF_ROWB), 0)
    ind = jnp.where(rows == mols, 1.0, 0.0).astype(_F32)
    hmol_ref[...] = _dot(ind, hatom)[None]


def _ffn(hnode, fnode, w1, w2, g, b):
    n = hnode.shape[0]
    grid = n // _FF_ROWB
    return pl.pallas_call(
        _ffn_body,
        grid=(grid,),
        in_specs=[
            pl.BlockSpec((_FF_ROWB, HSIZE), lambda i: (i, 0)),
            pl.BlockSpec((_FF_ROWB, NODE_FDIM), lambda i: (i, 0)),
            pl.BlockSpec((HSIZE + NODE_FDIM, 2 * HSIZE), lambda i: (0, 0)),
            pl.BlockSpec((2 * HSIZE, HSIZE), lambda i: (0, 0)),
            pl.BlockSpec((1, HSIZE), lambda i: (0, 0)),
            pl.BlockSpec((1, HSIZE), lambda i: (0, 0)),
        ],
        out_specs=[
            pl.BlockSpec((_FF_ROWB, HSIZE), lambda i: (i, 0)),
            pl.BlockSpec((1, _FF_MOLB, HSIZE), lambda i: (i, 0, 0)),
        ],
        out_shape=[
            jax.ShapeDtypeStruct((n, HSIZE), _F32),
            jax.ShapeDtypeStruct((grid, _FF_MOLB, HSIZE), _F32),
        ],
    )(hnode, fnode, w1, w2, g, b)


def _blockdiag(w):
    h, a, b = w.shape
    out = jnp.zeros((h * a, h * b), w.dtype)
    for i in range(h):
        out = out.at[i * a:(i + 1) * a, i * b:(i + 1) * b].set(w[i])
    return out


def kernel(fnode, fmess, agraph, bgraph, unused, atom_scope, bond_scope,
           W_i, W_z, U_z, W_r, U_r, W_h, U_h, W_o, W_ff1, W_ff2, ln_g, ln_b):
    bidx = jnp.pad(bgraph.reshape(-1).astype(jnp.int32),
                   (0, (E_PAD - N_EDGES) * MAX_NB))
    aidx = jnp.pad(agraph.reshape(-1).astype(jnp.int32),
                   (0, (A_PAD - N_NODES) * MAX_NB))
    fmess = jnp.pad(fmess, ((0, E_PAD - N_EDGES), (0, 0)))

    hnode = fnode
    for b in range(N_BLOCKS):
        wi_cat = jnp.concatenate([W_i[b, h] for h in range(N_HEADS)], axis=1)
        wzrh = jnp.concatenate(
            [_blockdiag(W_z[b]), _blockdiag(W_r[b]), _blockdiag(W_h[b])], axis=1)
        uzr = jnp.concatenate([_blockdiag(U_z[b]), _blockdiag(U_r[b])], axis=1)
        uh = _blockdiag(U_h[b])

        x, p = _mpn_init(fmess, wi_cat, wzrh)
        m = x
        for _ in range(DEPTH - 1):
            s_nei = _gsum_edges(m, bidx)
            m = _gru_update(s_nei, p, uzr, uh)

        nei = _gsum_atoms(m, aidx)[:N_NODES]
        wot = jnp.concatenate([W_o[b, h][:HSIZE] for h in range(N_HEADS)], axis=1)
        wob = _blockdiag(W_o[b, :, HSIZE:, :])
        hnode = _out_proj(hnode, nei, wot, wob)

    hatom, hmol = _ffn(hnode, fnode, W_ff1, W_ff2,
                       ln_g.reshape(1, -1), ln_b.reshape(1, -1))
    return (hmol.reshape(N_MOL, HSIZE), hatom)


# speedup vs baseline: 39.3613x; 1.0261x over previous
import functools

import jax
import jax.numpy as jnp
from jax import lax
from jax.experimental import pallas as pl
from jax.experimental.pallas import tpu as pltpu
from jax.experimental.pallas import tpu_sc as plsc

N_NODES = 10000
N_EDGES = 160000
MAX_NB = 16
NODE_FDIM = 128
EDGE_FDIM = 16
FMESS_DIM = NODE_FDIM + EDGE_FDIM
HSIZE = 128
N_HEADS = 4
HEAD = HSIZE // N_HEADS
DEPTH = 3
N_BLOCKS = 2
N_MOL = 100
A_PER = N_NODES // N_MOL

_F32 = jnp.float32
_NW = 32


_CH = 8
_NCH = 8
_NBUF = 4
_SUP = _CH * _NCH


@functools.lru_cache(maxsize=None)
def _make_gather_sum(n_out: int):
    epw = n_out // _NW
    assert epw * _NW == n_out and epw % _SUP == 0
    n_sup = epw // _SUP
    mesh = plsc.VectorSubcoreMesh(core_axis_name="c", subcore_axis_name="s")

    @functools.partial(
        pl.kernel,
        mesh=mesh,
        out_type=jax.ShapeDtypeStruct((n_out, HSIZE), _F32),
        scratch_types=[
            pltpu.VMEM((_SUP * MAX_NB,), jnp.int32),
            [pltpu.VMEM((_CH * MAX_NB, HSIZE), _F32) for _ in range(_NBUF)],
            pltpu.VMEM((_SUP, HSIZE), _F32),
            [pltpu.SemaphoreType.DMA for _ in range(_NBUF)],
        ],
    )
    def gsum(table_hbm, idx_hbm, out_hbm, idx_v, rows, acc_v, sems):
        wid = lax.axis_index("s") * 2 + lax.axis_index("c")
        base = wid * epw

        def start(c):
            return pltpu.async_copy(
                table_hbm.at[idx_v.at[pl.ds(c * _CH * MAX_NB, _CH * MAX_NB)]],
                rows[c % _NBUF], sems[c % _NBUF])

        def sup_body(sp, carry):
            e0 = base + sp * _SUP
            pltpu.sync_copy(idx_hbm.at[pl.ds(e0 * MAX_NB, _SUP * MAX_NB)], idx_v)
            cps = [None] * _NBUF
            for b in range(_NBUF - 1):
                cps[b] = start(b)
            for c in range(_NCH):
                nxt = c + _NBUF - 1
                if nxt < _NCH:
                    cps[nxt % _NBUF] = start(nxt)
                cps[c % _NBUF].wait()
                rv = rows[c % _NBUF]

                def edge_body(e, carry2, _c=c, _rv=rv):
                    r0 = e * MAX_NB
                    for s in range(HSIZE // 16):
                        col = pl.ds(s * 16, 16)
                        t = [_rv[r0 + 2 * j, col] + _rv[r0 + 2 * j + 1, col]
                             for j in range(8)]
                        u = [t[2 * j] + t[2 * j + 1] for j in range(4)]
                        acc_v[_c * _CH + e, col] = (u[0] + u[1]) + (u[2] + u[3])
                    return carry2

                lax.fori_loop(0, _CH, edge_body, 0, unroll=False)
            pltpu.sync_copy(acc_v, out_hbm.at[pl.ds(e0, _SUP)])
            return carry

        lax.fori_loop(0, n_sup, sup_body, 0, unroll=False)

    return gsum


E_PAD = 163840
A_PAD = 10240


def _gsum_edges(table, idx):
    return _make_gather_sum(E_PAD)(table, idx)


def _gsum_atoms(table, idx):
    return _make_gather_sum(A_PAD)(table, idx)


def _dot(a, b):
    return jnp.dot(a, b, preferred_element_type=_F32)


def _init_body(fmess_ref, wi_ref, wzrh_ref, x_ref, p_ref):
    x = jnp.maximum(_dot(fmess_ref[...], wi_ref[...]), 0.0)
    x_ref[...] = x
    p_ref[...] = _dot(x, wzrh_ref[...])


def _mpn_init(fmess, wi_cat, wzrh, rowb=2048):
    n = fmess.shape[0]
    grid = n // rowb
    return pl.pallas_call(
        _init_body,
        grid=(grid,),
        in_specs=[
            pl.BlockSpec((rowb, FMESS_DIM), lambda i: (i, 0)),
            pl.BlockSpec((FMESS_DIM, HSIZE), lambda i: (0, 0)),
            pl.BlockSpec((HSIZE, 3 * HSIZE), lambda i: (0, 0)),
        ],
        out_specs=[
            pl.BlockSpec((rowb, HSIZE), lambda i: (i, 0)),
            pl.BlockSpec((rowb, 3 * HSIZE), lambda i: (i, 0)),
        ],
        out_shape=[
            jax.ShapeDtypeStruct((n, HSIZE), _F32),
            jax.ShapeDtypeStruct((n, 3 * HSIZE), _F32),
        ],
    )(fmess, wi_cat, wzrh)


def _gru_body(s_ref, p_ref, uzr_ref, uh_ref, m_ref):
    s = s_ref[...]
    p = p_ref[...]
    q = _dot(s, uzr_ref[...])
    z = jax.nn.sigmoid(p[:, :HSIZE] + q[:, :HSIZE])
    r = jax.nn.sigmoid(p[:, HSIZE:2 * HSIZE] + q[:, HSIZE:])
    mt = jnp.tanh(p[:, 2 * HSIZE:] + _dot(r * s, uh_ref[...]))
    m_ref[...] = (1.0 - z) * s + z * mt


def _gru_update(s, p, uzr, uh, rowb=2048):
    n = s.shape[0]
    grid = n // rowb
    return pl.pallas_call(
        _gru_body,
        grid=(grid,),
        in_specs=[
            pl.BlockSpec((rowb, HSIZE), lambda i: (i, 0)),
            pl.BlockSpec((rowb, 3 * HSIZE), lambda i: (i, 0)),
            pl.BlockSpec((HSIZE, 2 * HSIZE), lambda i: (0, 0)),
            pl.BlockSpec((HSIZE, HSIZE), lambda i: (0, 0)),
        ],
        out_specs=pl.BlockSpec((rowb, HSIZE), lambda i: (i, 0)),
        out_shape=jax.ShapeDtypeStruct((n, HSIZE), _F32),
    )(s, p, uzr, uh)


def _outproj_body(hnode_ref, nei_ref, wot_ref, wob_ref, out_ref):
    out_ref[...] = jnp.maximum(
        _dot(hnode_ref[...], wot_ref[...]) + _dot(nei_ref[...], wob_ref[...]), 0.0)


def _out_proj(hnode, nei, wot, wob, rowb=2000):
    n = hnode.shape[0]
    grid = n // rowb
    return pl.pallas_call(
        _outproj_body,
        grid=(grid,),
        in_specs=[
            pl.BlockSpec((rowb, HSIZE), lambda i: (i, 0)),
            pl.BlockSpec((rowb, HSIZE), lambda i: (i, 0)),
            pl.BlockSpec((HSIZE, HSIZE), lambda i: (0, 0)),
            pl.BlockSpec((HSIZE, HSIZE), lambda i: (0, 0)),
        ],
        out_specs=pl.BlockSpec((rowb, HSIZE), lambda i: (i, 0)),
        out_shape=jax.ShapeDtypeStruct((n, HSIZE), _F32),
    )(hnode, nei, wot, wob)


_FF_ROWB = 2000
_FF_MOLB = _FF_ROWB // A_PER


def _ffn_body(hnode_ref, fnode_ref, w1_ref, w2_ref, g_ref, b_ref,
              hatom_ref, hmol_ref):
    hcat = jnp.concatenate([hnode_ref[...], fnode_ref[...]], axis=1)
    h1 = jnp.maximum(_dot(hcat, w1_ref[...]), 0.0)
    hmid = _dot(h1, w2_ref[...])
    mu = jnp.mean(hmid, axis=1, keepdims=True)
    var = jnp.mean((hmid - mu) ** 2, axis=1, keepdims=True)
    hatom = (hmid - mu) * lax.rsqrt(var + 1e-5) * g_ref[...] + b_ref[...]
    hatom_ref[...] = hatom
    rows = lax.broadcasted_iota(jnp.int32, (_FF_MOLB, _FF_ROWB), 1) // A_PER
    mols = lax.broadcasted_iota(jnp.int32, (_FF_MOLB, _FF_ROWB), 0)
    ind = jnp.where(rows == mols, 1.0, 0.0).astype(_F32)
    hmol_ref[...] = _dot(ind, hatom)[None]


def _ffn(hnode, fnode, w1, w2, g, b):
    n = hnode.shape[0]
    grid = n // _FF_ROWB
    return pl.pallas_call(
        _ffn_body,
        grid=(grid,),
        in_specs=[
            pl.BlockSpec((_FF_ROWB, HSIZE), lambda i: (i, 0)),
            pl.BlockSpec((_FF_ROWB, NODE_FDIM), lambda i: (i, 0)),
            pl.BlockSpec((HSIZE + NODE_FDIM, 2 * HSIZE), lambda i: (0, 0)),
            pl.BlockSpec((2 * HSIZE, HSIZE), lambda i: (0, 0)),
            pl.BlockSpec((1, HSIZE), lambda i: (0, 0)),
            pl.BlockSpec((1, HSIZE), lambda i: (0, 0)),
        ],
        out_specs=[
            pl.BlockSpec((_FF_ROWB, HSIZE), lambda i: (i, 0)),
            pl.BlockSpec((1, _FF_MOLB, HSIZE), lambda i: (i, 0, 0)),
        ],
        out_shape=[
            jax.ShapeDtypeStruct((n, HSIZE), _F32),
            jax.ShapeDtypeStruct((grid, _FF_MOLB, HSIZE), _F32),
        ],
    )(hnode, fnode, w1, w2, g, b)


def _blockdiag(w):
    h, a, b = w.shape
    out = jnp.zeros((h * a, h * b), w.dtype)
    for i in range(h):
        out = out.at[i * a:(i + 1) * a, i * b:(i + 1) * b].set(w[i])
    return out


def kernel(fnode, fmess, agraph, bgraph, unused, atom_scope, bond_scope,
           W_i, W_z, U_z, W_r, U_r, W_h, U_h, W_o, W_ff1, W_ff2, ln_g, ln_b):
    bidx = jnp.pad(bgraph.reshape(-1).astype(jnp.int32),
                   (0, (E_PAD - N_EDGES) * MAX_NB))
    aidx = jnp.pad(agraph.reshape(-1).astype(jnp.int32),
                   (0, (A_PAD - N_NODES) * MAX_NB))
    fmess = jnp.pad(fmess, ((0, E_PAD - N_EDGES), (0, 0)))

    hnode = fnode
    for b in range(N_BLOCKS):
        wi_cat = jnp.concatenate([W_i[b, h] for h in range(N_HEADS)], axis=1)
        wzrh = jnp.concatenate(
            [_blockdiag(W_z[b]), _blockdiag(W_r[b]), _blockdiag(W_h[b])], axis=1)
        uzr = jnp.concatenate([_blockdiag(U_z[b]), _blockdiag(U_r[b])], axis=1)
        uh = _blockdiag(U_h[b])

        x, p = _mpn_init(fmess, wi_cat, wzrh)
        m = x
        for _ in range(DEPTH - 1):
            s_nei = _gsum_edges(m, bidx)
            m = _gru_update(s_nei, p, uzr, uh)

        nei = _gsum_atoms(m, aidx)[:N_NODES]
        wot = jnp.concatenate([W_o[b, h][:HSIZE] for h in range(N_HEADS)], axis=1)
        wob = _blockdiag(W_o[b, :, HSIZE:, :])
        hnode = _out_proj(hnode, nei, wot, wob)

    hatom, hmol = _ffn(hnode, fnode, W_ff1, W_ff2,
                       ln_g.reshape(1, -1), ln_b.reshape(1, -1))
    return (hmol.reshape(N_MOL, HSIZE), hatom)
